# baseline (speedup 1.0000x reference)
import functools

import jax
import jax.numpy as jnp
from jax import lax
from jax.experimental import pallas as pl
from jax.experimental.pallas import tpu as pltpu
from jax.experimental.pallas import tpu_sc as plsc

NC = 2
NS = 16
NW = NC * NS
CHUNK = 128
NBUF = 2


def _make_deg_kernel(n_pad, steps):
    mesh = plsc.VectorSubcoreMesh(core_axis_name="c", subcore_axis_name="s")
    rows_per_tile = n_pad // NS
    zsteps = rows_per_tile // CHUNK

    @functools.partial(
        pl.kernel,
        out_type=jax.ShapeDtypeStruct((NC, n_pad, 16), jnp.float32),
        mesh=mesh,
        compiler_params=pltpu.CompilerParams(use_tc_tiling_on_sc=False),
        scratch_types=[
            pltpu.VMEM((steps, 2, CHUNK), jnp.int32),
            pltpu.VMEM((CHUNK, 16), jnp.float32),
            pltpu.VMEM((CHUNK, 16), jnp.float32),
            pltpu.VMEM_SHARED((n_pad, 16), jnp.float32),
        ],
    )
    def deg_kernel(ei_hbm, ones_hbm, zeros_hbm, out_hbm,
                   ei_v, ones_v, zeros_v, acc_sh):
        cid = lax.axis_index("c")
        sid = lax.axis_index("s")
        wid = cid * NS + sid
        pltpu.sync_copy(ones_hbm, ones_v)
        pltpu.sync_copy(zeros_hbm, zeros_v)
        pltpu.sync_copy(ei_hbm.at[pl.ds(wid * steps, steps)], ei_v)
        base = sid * rows_per_tile
        for j in range(zsteps):
            pltpu.sync_copy(zeros_v, acc_sh.at[pl.ds(base + j * CHUNK, CHUNK)])
        plsc.subcore_barrier()

        def body(k, _):
            pltpu.sync_copy(ones_v, acc_sh.at[ei_v.at[k, 1]], add=True)
            return ()

        lax.fori_loop(0, steps, body, ())
        plsc.subcore_barrier()
        pltpu.sync_copy(
            acc_sh.at[pl.ds(base, rows_per_tile)],
            out_hbm.at[cid, pl.ds(base, rows_per_tile)],
        )

    return deg_kernel


def _make_prop_kernel(n_pad, d, n0, n1):
    M = 2 * NBUF
    assert n0 % M == 0 and n1 % M == 0
    mesh = plsc.VectorSubcoreMesh(core_axis_name="c", subcore_axis_name="s")
    rows_per_tile = n_pad // NS
    zsteps = rows_per_tile // CHUNK

    @functools.partial(
        pl.kernel,
        out_type=jax.ShapeDtypeStruct((NC, n_pad, d), jnp.float32),
        mesh=mesh,
        compiler_params=pltpu.CompilerParams(use_tc_tiling_on_sc=False),
        scratch_types=[
            [pltpu.VMEM((2, CHUNK), jnp.int32)] * M,
            [pltpu.VMEM((CHUNK, d), jnp.float32)] * NBUF,
            pltpu.VMEM_SHARED((n_pad, d), jnp.float32),
            [pltpu.SemaphoreType.DMA] * M,
            [pltpu.SemaphoreType.DMA] * NBUF,
        ],
    )
    def prop_kernel(ei_hbm, g_hbm, zeros_hbm, out_hbm,
                    idx_v, rows_v, acc_sh, isems, gsems):
        cid = lax.axis_index("c")
        sid = lax.axis_index("s")
        nc = jnp.where(cid == 0, n0, n1)
        iters = jnp.where(cid == 0, n0 // M, n1 // M)
        erow = jnp.where(cid == 0, sid * n0, NS * n0 + sid * n1)
        pltpu.sync_copy(zeros_hbm, rows_v[0])
        base = sid * rows_per_tile
        for j in range(zsteps):
            pltpu.sync_copy(rows_v[0],
                            acc_sh.at[pl.ds(base + j * CHUNK, CHUNK)])
        plsc.subcore_barrier()

        for j in range(M):
            pltpu.async_copy(ei_hbm.at[erow + j], idx_v[j], isems[j])
        for b in range(NBUF):
            pltpu.make_async_copy(ei_hbm.at[0], idx_v[b], isems[b]).wait()
            pltpu.async_copy(g_hbm.at[idx_v[b].at[0]], rows_v[b], gsems[b])

        def body(i, _):
            for s in range(M):
                b = s % NBUF
                k = i * M + s
                pltpu.make_async_copy(g_hbm.at[pl.ds(0, CHUNK)],
                                      rows_v[b], gsems[b]).wait()
                pltpu.sync_copy(rows_v[b], acc_sh.at[idx_v[s].at[1]],
                                add=True)
                nk = lax.rem(k + M, nc)
                pltpu.async_copy(ei_hbm.at[erow + nk], idx_v[s], isems[s])
                s2 = (s + NBUF) % M
                pltpu.make_async_copy(ei_hbm.at[0], idx_v[s2],
                                      isems[s2]).wait()
                pltpu.async_copy(g_hbm.at[idx_v[s2].at[0]], rows_v[b],
                                 gsems[b])
            return ()

        lax.fori_loop(0, iters, body, ())
        for b in range(NBUF):
            pltpu.make_async_copy(g_hbm.at[pl.ds(0, CHUNK)],
                                  rows_v[b], gsems[b]).wait()
        for s in range(M - NBUF, M):
            pltpu.make_async_copy(ei_hbm.at[0], idx_v[s], isems[s]).wait()
        plsc.subcore_barrier()
        pltpu.sync_copy(
            acc_sh.at[pl.ds(base, rows_per_tile)],
            out_hbm.at[cid, pl.ds(base, rows_per_tile)],
        )

    return prop_kernel


def _dis_from_deg(da, db):
    deg = da[:, :1] + db[:, :1] + 1.0
    return lax.rsqrt(deg)


def _tc_h1(x, w1, n_pad, d_in, d_hid, blk):
    def body(x_ref, w_ref, h_ref):
        h_ref[...] = jnp.dot(x_ref[...], w_ref[...],
                             preferred_element_type=jnp.float32)

    return pl.pallas_call(
        body,
        grid=(n_pad // blk,),
        in_specs=[
            pl.BlockSpec((blk, d_in), lambda i: (i, 0)),
            pl.BlockSpec((d_in, d_hid), lambda i: (0, 0)),
        ],
        out_specs=pl.BlockSpec((blk, d_hid), lambda i: (i, 0)),
        out_shape=jax.ShapeDtypeStruct((n_pad, d_hid), jnp.float32),
    )(x, w1)


def _tc_scale(dega, degb, h1, n_pad, d_hid, blk):
    def body(da_ref, db_ref, h_ref, g_ref):
        dis = _dis_from_deg(da_ref[...], db_ref[...])
        g_ref[...] = dis * h_ref[...]

    return pl.pallas_call(
        body,
        grid=(n_pad // blk,),
        in_specs=[
            pl.BlockSpec((blk, 16), lambda i: (i, 0)),
            pl.BlockSpec((blk, 16), lambda i: (i, 0)),
            pl.BlockSpec((blk, d_hid), lambda i: (i, 0)),
        ],
        out_specs=pl.BlockSpec((blk, d_hid), lambda i: (i, 0)),
        out_shape=jax.ShapeDtypeStruct((n_pad, d_hid), jnp.float32),
    )(dega, degb, h1)


def _tc_mid(u1, dega, degb, g1, b1, w2, n_pad, d_hid, d_out, blk):
    def body(ua_ref, ub_ref, da_ref, db_ref, g_ref, b_ref, w_ref, o_ref):
        dis = _dis_from_deg(da_ref[...], db_ref[...])
        z = jax.nn.relu(dis * (ua_ref[0] + ub_ref[0] + g_ref[...])
                        + b_ref[...])
        o_ref[...] = dis * jnp.dot(z, w_ref[...],
                                   preferred_element_type=jnp.float32)

    return pl.pallas_call(
        body,
        grid=(n_pad // blk,),
        in_specs=[
            pl.BlockSpec((1, blk, d_hid), lambda i: (0, i, 0)),
            pl.BlockSpec((1, blk, d_hid), lambda i: (1, i, 0)),
            pl.BlockSpec((blk, 16), lambda i: (i, 0)),
            pl.BlockSpec((blk, 16), lambda i: (i, 0)),
            pl.BlockSpec((blk, d_hid), lambda i: (i, 0)),
            pl.BlockSpec((1, d_hid), lambda i: (0, 0)),
            pl.BlockSpec((d_hid, d_out), lambda i: (0, 0)),
        ],
        out_specs=pl.BlockSpec((blk, d_out), lambda i: (i, 0)),
        out_shape=jax.ShapeDtypeStruct((n_pad, d_out), jnp.float32),
    )(u1, u1, dega, degb, g1, b1, w2)


def _tc_out(u2, dega, degb, g2, b2, n_pad, d_out, blk):
    def body(ua_ref, ub_ref, da_ref, db_ref, g_ref, b_ref, o_ref):
        dis = _dis_from_deg(da_ref[...], db_ref[...])
        o_ref[...] = jax.nn.sigmoid(
            dis * (ua_ref[0] + ub_ref[0] + g_ref[...]) + b_ref[...])

    return pl.pallas_call(
        body,
        grid=(n_pad // blk,),
        in_specs=[
            pl.BlockSpec((1, blk, d_out), lambda i: (0, i, 0)),
            pl.BlockSpec((1, blk, d_out), lambda i: (1, i, 0)),
            pl.BlockSpec((blk, 16), lambda i: (i, 0)),
            pl.BlockSpec((blk, 16), lambda i: (i, 0)),
            pl.BlockSpec((blk, d_out), lambda i: (i, 0)),
            pl.BlockSpec((1, d_out), lambda i: (0, 0)),
        ],
        out_specs=pl.BlockSpec((blk, d_out), lambda i: (i, 0)),
        out_shape=jax.ShapeDtypeStruct((n_pad, d_out), jnp.float32),
    )(u2, u2, dega, degb, g2, b2)


def kernel(x, edge_index, W1, b1, W2, b2):
    n, d_in = x.shape
    d_hid = W1.shape[1]
    d_out = W2.shape[1]
    e = edge_index.shape[1]

    n_pad = ((n + CHUNK * NS - 1) // (CHUNK * NS)) * (CHUNK * NS)
    steps = (e + NW * CHUNK - 1) // (NW * CHUNK)
    steps = ((steps + 7) // 8) * 8
    e_pad = NW * CHUNK * steps
    t_pair = 2 * steps
    n0 = t_pair // 2
    n1 = t_pair - n0

    src = edge_index[0]
    dst = edge_index[1]
    pad_e = e_pad - e
    dump = n + (jnp.arange(pad_e, dtype=jnp.int32) % (n_pad - n))
    src_p = jnp.concatenate([src, dump]).reshape(e_pad // CHUNK, 1, CHUNK)
    dst_p = jnp.concatenate([dst, dump]).reshape(e_pad // CHUNK, 1, CHUNK)
    ei_p = jnp.concatenate([src_p, dst_p], axis=1)

    x_p = jnp.zeros((n_pad, d_in), x.dtype).at[:n].set(x)
    ones_rows = jnp.ones((CHUNK, 16), jnp.float32)
    zeros16 = jnp.zeros((CHUNK, 16), jnp.float32)
    zeros_hid = jnp.zeros((CHUNK, d_hid), jnp.float32)
    zeros_out = jnp.zeros((CHUNK, d_out), jnp.float32)

    blk = 1024
    deg16 = _make_deg_kernel(n_pad, steps)(ei_p, ones_rows, zeros16)
    h1 = _tc_h1(x_p, W1, n_pad, d_in, d_hid, blk)
    dega, degb = deg16[0], deg16[1]
    g1 = _tc_scale(dega, degb, h1, n_pad, d_hid, blk)

    u1 = _make_prop_kernel(n_pad, d_hid, n0, n1)(ei_p, g1, zeros_hid)

    g2 = _tc_mid(u1, dega, degb, g1, b1.reshape(1, d_hid), W2,
                 n_pad, d_hid, d_out, blk)

    u2 = _make_prop_kernel(n_pad, d_out, n0, n1)(ei_p, g2, zeros_out)

    out = _tc_out(u2, dega, degb, g2, b2.reshape(1, d_out), n_pad, d_out, blk)
    return out[:n]


# speedup vs baseline: 1.0546x; 1.0546x over previous
import functools

import jax
import jax.numpy as jnp
from jax import lax
from jax.experimental import pallas as pl
from jax.experimental.pallas import tpu as pltpu
from jax.experimental.pallas import tpu_sc as plsc

NC = 2
NS = 16
NW = NC * NS
CHUNK = 112
NBUF = 3


def _make_deg_kernel(n_pad, steps):
    mesh = plsc.VectorSubcoreMesh(core_axis_name="c", subcore_axis_name="s")
    rows_per_tile = n_pad // NS
    zsteps = rows_per_tile // CHUNK

    @functools.partial(
        pl.kernel,
        out_type=jax.ShapeDtypeStruct((NC, n_pad, 16), jnp.float32),
        mesh=mesh,
        compiler_params=pltpu.CompilerParams(use_tc_tiling_on_sc=False),
        scratch_types=[
            pltpu.VMEM((steps, 2, CHUNK), jnp.int32),
            pltpu.VMEM((CHUNK, 16), jnp.float32),
            pltpu.VMEM((CHUNK, 16), jnp.float32),
            pltpu.VMEM_SHARED((n_pad, 16), jnp.float32),
        ],
    )
    def deg_kernel(ei_hbm, ones_hbm, zeros_hbm, out_hbm,
                   ei_v, ones_v, zeros_v, acc_sh):
        cid = lax.axis_index("c")
        sid = lax.axis_index("s")
        wid = cid * NS + sid
        pltpu.sync_copy(ones_hbm, ones_v)
        pltpu.sync_copy(zeros_hbm, zeros_v)
        pltpu.sync_copy(ei_hbm.at[pl.ds(wid * steps, steps)], ei_v)
        base = sid * rows_per_tile
        for j in range(zsteps):
            pltpu.sync_copy(zeros_v, acc_sh.at[pl.ds(base + j * CHUNK, CHUNK)])
        zrem = rows_per_tile % CHUNK
        if zrem:
            pltpu.sync_copy(
                zeros_v.at[pl.ds(0, zrem)],
                acc_sh.at[pl.ds(base + zsteps * CHUNK, zrem)])
        plsc.subcore_barrier()

        def body(k, _):
            pltpu.sync_copy(ones_v, acc_sh.at[ei_v.at[k, 1]], add=True)
            return ()

        lax.fori_loop(0, steps, body, ())
        plsc.subcore_barrier()
        pltpu.sync_copy(
            acc_sh.at[pl.ds(base, rows_per_tile)],
            out_hbm.at[cid, pl.ds(base, rows_per_tile)],
        )

    return deg_kernel


def _make_prop_kernel(n_pad, d, n0, n1):
    M = 2 * NBUF
    assert n0 % M == 0 and n1 % M == 0
    mesh = plsc.VectorSubcoreMesh(core_axis_name="c", subcore_axis_name="s")
    rows_per_tile = n_pad // NS
    zsteps = rows_per_tile // CHUNK

    @functools.partial(
        pl.kernel,
        out_type=jax.ShapeDtypeStruct((NC, n_pad, d), jnp.float32),
        mesh=mesh,
        compiler_params=pltpu.CompilerParams(use_tc_tiling_on_sc=False),
        scratch_types=[
            [pltpu.VMEM((2, CHUNK), jnp.int32)] * M,
            [pltpu.VMEM((CHUNK, d), jnp.float32)] * NBUF,
            pltpu.VMEM_SHARED((n_pad, d), jnp.float32),
            [pltpu.SemaphoreType.DMA] * M,
            [pltpu.SemaphoreType.DMA] * NBUF,
        ],
    )
    def prop_kernel(ei_hbm, g_hbm, zeros_hbm, out_hbm,
                    idx_v, rows_v, acc_sh, isems, gsems):
        cid = lax.axis_index("c")
        sid = lax.axis_index("s")
        nc = jnp.where(cid == 0, n0, n1)
        iters = jnp.where(cid == 0, n0 // M, n1 // M)
        erow = jnp.where(cid == 0, sid * n0, NS * n0 + sid * n1)
        pltpu.sync_copy(zeros_hbm, rows_v[0])
        base = sid * rows_per_tile
        for j in range(zsteps):
            pltpu.sync_copy(rows_v[0],
                            acc_sh.at[pl.ds(base + j * CHUNK, CHUNK)])
        zrem = rows_per_tile % CHUNK
        if zrem:
            pltpu.sync_copy(
                rows_v[0].at[pl.ds(0, zrem)],
                acc_sh.at[pl.ds(base + zsteps * CHUNK, zrem)])
        plsc.subcore_barrier()

        for j in range(M):
            pltpu.async_copy(ei_hbm.at[erow + j], idx_v[j], isems[j])
        for b in range(NBUF):
            pltpu.make_async_copy(ei_hbm.at[0], idx_v[b], isems[b]).wait()
            pltpu.async_copy(g_hbm.at[idx_v[b].at[0]], rows_v[b], gsems[b])

        def body(i, _):
            for s in range(M):
                b = s % NBUF
                k = i * M + s
                pltpu.make_async_copy(g_hbm.at[pl.ds(0, CHUNK)],
                                      rows_v[b], gsems[b]).wait()
                pltpu.sync_copy(rows_v[b], acc_sh.at[idx_v[s].at[1]],
                                add=True)
                nk = lax.rem(k + M, nc)
                pltpu.async_copy(ei_hbm.at[erow + nk], idx_v[s], isems[s])
                s2 = (s + NBUF) % M
                pltpu.make_async_copy(ei_hbm.at[0], idx_v[s2],
                                      isems[s2]).wait()
                pltpu.async_copy(g_hbm.at[idx_v[s2].at[0]], rows_v[b],
                                 gsems[b])
            return ()

        lax.fori_loop(0, iters, body, ())
        for b in range(NBUF):
            pltpu.make_async_copy(g_hbm.at[pl.ds(0, CHUNK)],
                                  rows_v[b], gsems[b]).wait()
        for s in range(M - NBUF, M):
            pltpu.make_async_copy(ei_hbm.at[0], idx_v[s], isems[s]).wait()
        plsc.subcore_barrier()
        pltpu.sync_copy(
            acc_sh.at[pl.ds(base, rows_per_tile)],
            out_hbm.at[cid, pl.ds(base, rows_per_tile)],
        )

    return prop_kernel


def _dis_from_deg(da, db):
    deg = da[:, :1] + db[:, :1] + 1.0
    return lax.rsqrt(deg)


def _tc_h1(x, w1, n_pad, d_in, d_hid, blk):
    def body(x_ref, w_ref, h_ref):
        h_ref[...] = jnp.dot(x_ref[...], w_ref[...],
                             preferred_element_type=jnp.float32)

    return pl.pallas_call(
        body,
        grid=(n_pad // blk,),
        in_specs=[
            pl.BlockSpec((blk, d_in), lambda i: (i, 0)),
            pl.BlockSpec((d_in, d_hid), lambda i: (0, 0)),
        ],
        out_specs=pl.BlockSpec((blk, d_hid), lambda i: (i, 0)),
        out_shape=jax.ShapeDtypeStruct((n_pad, d_hid), jnp.float32),
    )(x, w1)


def _tc_scale(dega, degb, h1, n_pad, d_hid, blk):
    def body(da_ref, db_ref, h_ref, g_ref):
        dis = _dis_from_deg(da_ref[...], db_ref[...])
        g_ref[...] = dis * h_ref[...]

    return pl.pallas_call(
        body,
        grid=(n_pad // blk,),
        in_specs=[
            pl.BlockSpec((blk, 16), lambda i: (i, 0)),
            pl.BlockSpec((blk, 16), lambda i: (i, 0)),
            pl.BlockSpec((blk, d_hid), lambda i: (i, 0)),
        ],
        out_specs=pl.BlockSpec((blk, d_hid), lambda i: (i, 0)),
        out_shape=jax.ShapeDtypeStruct((n_pad, d_hid), jnp.float32),
    )(dega, degb, h1)


def _tc_mid(u1, dega, degb, g1, b1, w2, n_pad, d_hid, d_out, blk):
    def body(ua_ref, ub_ref, da_ref, db_ref, g_ref, b_ref, w_ref, o_ref):
        dis = _dis_from_deg(da_ref[...], db_ref[...])
        z = jax.nn.relu(dis * (ua_ref[0] + ub_ref[0] + g_ref[...])
                        + b_ref[...])
        o_ref[...] = dis * jnp.dot(z, w_ref[...],
                                   preferred_element_type=jnp.float32)

    return pl.pallas_call(
        body,
        grid=(n_pad // blk,),
        in_specs=[
            pl.BlockSpec((1, blk, d_hid), lambda i: (0, i, 0)),
            pl.BlockSpec((1, blk, d_hid), lambda i: (1, i, 0)),
            pl.BlockSpec((blk, 16), lambda i: (i, 0)),
            pl.BlockSpec((blk, 16), lambda i: (i, 0)),
            pl.BlockSpec((blk, d_hid), lambda i: (i, 0)),
            pl.BlockSpec((1, d_hid), lambda i: (0, 0)),
            pl.BlockSpec((d_hid, d_out), lambda i: (0, 0)),
        ],
        out_specs=pl.BlockSpec((blk, d_out), lambda i: (i, 0)),
        out_shape=jax.ShapeDtypeStruct((n_pad, d_out), jnp.float32),
    )(u1, u1, dega, degb, g1, b1, w2)


def _tc_out(u2, dega, degb, g2, b2, n_pad, d_out, blk):
    def body(ua_ref, ub_ref, da_ref, db_ref, g_ref, b_ref, o_ref):
        dis = _dis_from_deg(da_ref[...], db_ref[...])
        o_ref[...] = jax.nn.sigmoid(
            dis * (ua_ref[0] + ub_ref[0] + g_ref[...]) + b_ref[...])

    return pl.pallas_call(
        body,
        grid=(n_pad // blk,),
        in_specs=[
            pl.BlockSpec((1, blk, d_out), lambda i: (0, i, 0)),
            pl.BlockSpec((1, blk, d_out), lambda i: (1, i, 0)),
            pl.BlockSpec((blk, 16), lambda i: (i, 0)),
            pl.BlockSpec((blk, 16), lambda i: (i, 0)),
            pl.BlockSpec((blk, d_out), lambda i: (i, 0)),
            pl.BlockSpec((1, d_out), lambda i: (0, 0)),
        ],
        out_specs=pl.BlockSpec((blk, d_out), lambda i: (i, 0)),
        out_shape=jax.ShapeDtypeStruct((n_pad, d_out), jnp.float32),
    )(u2, u2, dega, degb, g2, b2)


def kernel(x, edge_index, W1, b1, W2, b2):
    n, d_in = x.shape
    d_hid = W1.shape[1]
    d_out = W2.shape[1]
    e = edge_index.shape[1]

    n_pad = ((n + 128 * NS - 1) // (128 * NS)) * (128 * NS)
    m = 2 * NBUF
    steps = (e + NW * CHUNK - 1) // (NW * CHUNK)
    steps = ((steps + m - 1) // m) * m
    e_pad = NW * CHUNK * steps
    t_pair = 2 * steps
    n0 = t_pair // 2
    n1 = t_pair - n0

    src = edge_index[0]
    dst = edge_index[1]
    pad_e = e_pad - e
    dump = n + (jnp.arange(pad_e, dtype=jnp.int32) % (n_pad - n))
    src_p = jnp.concatenate([src, dump]).reshape(e_pad // CHUNK, 1, CHUNK)
    dst_p = jnp.concatenate([dst, dump]).reshape(e_pad // CHUNK, 1, CHUNK)
    ei_p = jnp.concatenate([src_p, dst_p], axis=1)

    x_p = jnp.zeros((n_pad, d_in), x.dtype).at[:n].set(x)
    ones_rows = jnp.ones((CHUNK, 16), jnp.float32)
    zeros16 = jnp.zeros((CHUNK, 16), jnp.float32)
    zeros_hid = jnp.zeros((CHUNK, d_hid), jnp.float32)
    zeros_out = jnp.zeros((CHUNK, d_out), jnp.float32)

    blk = 1024
    deg16 = _make_deg_kernel(n_pad, steps)(ei_p, ones_rows, zeros16)
    h1 = _tc_h1(x_p, W1, n_pad, d_in, d_hid, blk)
    dega, degb = deg16[0], deg16[1]
    g1 = _tc_scale(dega, degb, h1, n_pad, d_hid, blk)

    u1 = _make_prop_kernel(n_pad, d_hid, n0, n1)(ei_p, g1, zeros_hid)

    g2 = _tc_mid(u1, dega, degb, g1, b1.reshape(1, d_hid), W2,
                 n_pad, d_hid, d_out, blk)

    u2 = _make_prop_kernel(n_pad, d_out, n0, n1)(ei_p, g2, zeros_out)

    out = _tc_out(u2, dega, degb, g2, b2.reshape(1, d_out), n_pad, d_out, blk)
    return out[:n]
